# baseline (speedup 1.0000x reference)
import numpy as np
import jax
import jax.numpy as jnp
from jax import lax
from jax.experimental import pallas as pl
from jax.experimental.pallas import tpu as pltpu
from jax.experimental.pallas import tpu_sc as plsc

N = 1048576
K = 52428
NT = 16
EPT = N // NT
VCH = 8192
NCH = EPT // VCH
L = 16

MIN32 = np.int32(-(2 ** 31))


def _fwd_key(v):
    b = lax.bitcast_convert_type(v, jnp.int32)
    m = b >> 31
    return lax.bitcast_convert_type(b ^ (m | MIN32), jnp.uint32)


def _inv_val(key):
    ki = lax.bitcast_convert_type(key, jnp.int32)
    s = ki >> 31
    b = ki ^ ((s & MIN32) | ~s)
    return lax.bitcast_convert_type(b, jnp.float32)


def _tail_body(x_hbm, out_hbm, keybuf, hcnt, red_cnt, stg, outv, sball,
               dsem, shr_cnt, shr_sum):
    sid = lax.axis_index("s")
    lane = lax.iota(jnp.int32, 16)
    ones = jnp.ones((L,), jnp.int32)
    zi = jnp.zeros((L,), jnp.int32)
    zf = jnp.zeros((L,), jnp.float32)

    def zero_hist():
        def zh(j):
            hcnt[pl.ds(j * L, L)] = zi
        plsc.parallel_loop(0, 256, unroll=8)(zh)

    def publish_and_decide(r, kprime):
        def lrj(j, _):
            def lr(l, acc):
                return acc + hcnt[pl.ds(l * 256 + j * L, L)]
            ac = lax.fori_loop(0, 16, lr, zi)
            red_cnt[pl.ds(j * L, L)] = ac
            return 0
        lax.fori_loop(0, 16, lrj, 0)
        pds = [
            pltpu.async_copy(
                red_cnt.at[pl.ds(j * L, L)],
                shr_cnt.at[pl.ds(r * 4096 + j * 256 + sid * L, L)],
                dsem.at[j])
            for j in range(16)
        ]
        for d in pds:
            d.wait()
        plsc.subcore_barrier()
        pltpu.sync_copy(shr_cnt.at[pl.ds(r * 4096 + sid * 256, 256)],
                        red_cnt)

        def rb(t, acc):
            return acc + red_cnt[pl.ds(t * L, L)]
        stg[...] = lax.fori_loop(0, 16, rb, zi)
        pltpu.sync_copy(stg, shr_cnt.at[pl.ds(16384 + r * 256 + sid * L, L)])
        plsc.subcore_barrier()
        pltpu.sync_copy(shr_cnt.at[pl.ds(16384 + r * 256, 256)], red_cnt)

        def dj(j, st):
            carry, bacc, cntb = st
            cvec = red_cnt[pl.ds(j * L, L)]
            cum = plsc.cumsum(cvec) + carry
            m = cum < kprime
            bacc = bacc + jnp.sum(jnp.where(m, ones, zi))
            cntb = cntb + jnp.sum(jnp.where(m, cvec, zi))
            carry = carry + jnp.sum(cvec)
            return carry, bacc, cntb
        _, bacc, cntb = lax.fori_loop(
            0, 16, dj, (jnp.int32(0), jnp.int32(0), jnp.int32(0)))
        return bacc, kprime - cntb

    zero_hist()
    base = sid * EPT
    descs = [
        pltpu.async_copy(x_hbm.at[pl.ds(base + c * VCH, VCH)],
                         keybuf.at[pl.ds(c * VCH, VCH)], dsem.at[c])
        for c in range(NCH)
    ]
    for c in range(NCH):
        descs[c].wait()

        def r0(i, c=c):
            v = keybuf[pl.ds(c * VCH + i * L, L)]
            key = _fwd_key(v)
            byte = (key >> jnp.uint32(24)).astype(jnp.int32)
            plsc.addupdate_scatter(hcnt, [lane * 256 + byte], ones)
        plsc.parallel_loop(0, VCH // L, unroll=8)(r0)

    kprime = jnp.int32(K)
    b0, kprime = publish_and_decide(0, kprime)
    prefix = b0.astype(jnp.uint32)
    sumb = zf

    SEG = 16
    SEGSZ = EPT // SEG
    zero_hist()
    thr1 = prefix << jnp.uint32(24)

    def r1(b, st, prefix=prefix, thr1=thr1):
        wps, sb = st
        new_wps = []
        for s in range(SEG):
            kf = keybuf[pl.ds(s * SEGSZ + b * L, L)]
            key = _fwd_key(kf)
            act = (key >> jnp.uint32(24)) == prefix
            below = key < thr1
            byte = ((key >> jnp.uint32(16))
                    & jnp.uint32(0xFF)).astype(jnp.int32)
            plsc.addupdate_scatter(hcnt, [lane * 256 + byte], ones,
                                   mask=act)
            sb = sb + jnp.where(below, kf, zf)
            plsc.store_compressed(keybuf.at[pl.ds(wps[s], L)], kf, mask=act)
            new_wps.append(wps[s] + plsc.all_reduce_population_count(act)[0])
        return tuple(new_wps), sb

    wps0 = tuple(jnp.int32(s * SEGSZ) for s in range(SEG))
    wps_f, sumb = plsc.parallel_loop(
        0, SEGSZ // L, carry=(wps0, sumb))(r1)
    b1, kprime = publish_and_decide(1, kprime)
    prefix = prefix * jnp.uint32(256) + b1.astype(jnp.uint32)

    zero_hist()
    thr2 = prefix << jnp.uint32(16)
    wp2 = jnp.int32(0)
    sb2 = sumb
    for s in range(SEG):
        ns = wps_f[s] - jnp.int32(s * SEGSZ)
        nch_s = (ns + L - 1) >> 4

        def r2b(i, st, s=s, ns=ns, prefix=prefix, thr2=thr2):
            wp, sb = st
            kf = keybuf[pl.ds(s * SEGSZ + i * L, L)]
            key = _fwd_key(kf)
            valid = (i * L + lane) < ns
            act = valid & ((key >> jnp.uint32(16)) == prefix)
            below = valid & (key < thr2)
            byte = ((key >> jnp.uint32(8))
                    & jnp.uint32(0xFF)).astype(jnp.int32)
            plsc.addupdate_scatter(hcnt, [lane * 256 + byte], ones,
                                   mask=act)
            sb = sb + jnp.where(below, kf, zf)
            plsc.store_compressed(keybuf.at[pl.ds(wp, L)], kf, mask=act)
            wp = wp + plsc.all_reduce_population_count(act)[0]
            return wp, sb

        wp2, sb2 = lax.fori_loop(0, nch_s, r2b, (wp2, sb2))
    sumb = sb2
    b2, kprime = publish_and_decide(2, kprime)
    prefix = prefix * jnp.uint32(256) + b2.astype(jnp.uint32)
    n = wp2

    zero_hist()
    thr3 = prefix << jnp.uint32(8)
    nch = (n + L - 1) >> 4

    def r3b(i, st, prefix=prefix, thr3=thr3, n=n):
        wp, sb = st
        kf = keybuf[pl.ds(i * L, L)]
        key = _fwd_key(kf)
        valid = (i * L + lane) < n
        act = valid & ((key >> jnp.uint32(8)) == prefix)
        below = valid & (key < thr3)
        byte = (key & jnp.uint32(0xFF)).astype(jnp.int32)
        plsc.addupdate_scatter(hcnt, [lane * 256 + byte], ones, mask=act)
        sb = sb + jnp.where(below, kf, zf)
        plsc.store_compressed(keybuf.at[pl.ds(wp, L)], kf, mask=act)
        wp = wp + plsc.all_reduce_population_count(act)[0]
        return wp, sb

    n, sumb = lax.fori_loop(0, nch, r3b, (jnp.int32(0), sumb))
    b3, kprime = publish_and_decide(3, kprime)
    prefix = prefix * jnp.uint32(256) + b3.astype(jnp.uint32)

    nch = (n + L - 1) >> 4

    def fbody(i, sb, prefix=prefix, n=n):
        kf = keybuf[pl.ds(i * L, L)]
        key = _fwd_key(kf)
        below = ((i * L + lane) < n) & (key < prefix)
        return sb + jnp.where(below, kf, zf)

    sumb = lax.fori_loop(0, nch, fbody, sumb)

    outv[...] = sumb
    pltpu.sync_copy(outv, shr_sum.at[pl.ds(sid * L, L)])
    plsc.subcore_barrier()

    @pl.when(sid == 0)
    def _():
        pltpu.sync_copy(shr_sum, sball)

        def sred(t, acc):
            return acc + sball[pl.ds(t * L, L)]
        sacc = lax.fori_loop(0, 16, sred, zf)
        tval = _inv_val(jnp.full((L,), prefix, jnp.uint32))
        total_below = jnp.sum(sacc)
        res = (total_below
               + kprime.astype(jnp.float32) * tval) / jnp.float32(K)
        outv[...] = res
        pltpu.sync_copy(outv, out_hbm)


def _build():
    mesh = plsc.VectorSubcoreMesh(
        core_axis_name="c", subcore_axis_name="s", num_cores=1,
        num_subcores=NT)
    return pl.kernel(
        _tail_body,
        out_type=jax.ShapeDtypeStruct((L,), jnp.float32),
        mesh=mesh,
        scratch_types=[
            pltpu.VMEM((EPT,), jnp.float32),
            pltpu.VMEM((16 * 256,), jnp.int32),
            pltpu.VMEM((256,), jnp.int32),
            pltpu.VMEM((L,), jnp.int32),
            pltpu.VMEM((L,), jnp.float32),
            pltpu.VMEM((16 * L,), jnp.float32),
            pltpu.SemaphoreType.DMA((16,)),
            pltpu.VMEM_SHARED((4 * 4096 + 4 * 256,), jnp.int32),
            pltpu.VMEM_SHARED((16 * L,), jnp.float32),
        ],
        compiler_params=pltpu.CompilerParams(needs_layout_passes=False),
    )


_tail_kernel = None


def kernel(portfolio_value):
    global _tail_kernel
    if _tail_kernel is None:
        _tail_kernel = jax.jit(_build())
    return _tail_kernel(portfolio_value)[0]


# speedup vs baseline: 1.2196x; 1.2196x over previous
import numpy as np
import jax
import jax.numpy as jnp
from jax import lax
from jax.experimental import pallas as pl
from jax.experimental.pallas import tpu as pltpu
from jax.experimental.pallas import tpu_sc as plsc

N = 1048576
K = 52428
NT = 16
EPT = N // NT
VCH = 8192
NCH = EPT // VCH
L = 16

MIN32 = np.int32(-(2 ** 31))


def _fwd_key(v):
    b = lax.bitcast_convert_type(v, jnp.int32)
    m = b >> 31
    return lax.bitcast_convert_type(b ^ (m | MIN32), jnp.uint32)


def _inv_val(key):
    ki = lax.bitcast_convert_type(key, jnp.int32)
    s = ki >> 31
    b = ki ^ ((s & MIN32) | ~s)
    return lax.bitcast_convert_type(b, jnp.float32)


def _tail_body(x_hbm, out_hbm, keybuf, hcnt, red_cnt, stg, outv, sball,
               dsem, shr_cnt, shr_sum):
    sid = lax.axis_index("s")
    lane = lax.iota(jnp.int32, 16)
    ones = jnp.ones((L,), jnp.int32)
    zi = jnp.zeros((L,), jnp.int32)
    zf = jnp.zeros((L,), jnp.float32)

    def zero_hist():
        def zh(j):
            hcnt[pl.ds(j * L, L)] = zi
        plsc.parallel_loop(0, 256, unroll=8)(zh)

    def publish_and_decide(r, kprime):
        def lrj(j, _):
            def lr(l, acc):
                return acc + hcnt[pl.ds(l * 256 + j * L, L)]
            ac = lax.fori_loop(0, 16, lr, zi)
            red_cnt[pl.ds(j * L, L)] = ac
            return 0
        lax.fori_loop(0, 16, lrj, 0)
        pds = [
            pltpu.async_copy(
                red_cnt.at[pl.ds(j * L, L)],
                shr_cnt.at[pl.ds(r * 4096 + j * 256 + sid * L, L)],
                dsem.at[j])
            for j in range(16)
        ]
        for d in pds:
            d.wait()
        plsc.subcore_barrier()
        pltpu.sync_copy(shr_cnt.at[pl.ds(r * 4096 + sid * 256, 256)],
                        red_cnt)

        def rb(t, acc):
            return acc + red_cnt[pl.ds(t * L, L)]
        stg[...] = lax.fori_loop(0, 16, rb, zi)
        pltpu.sync_copy(stg, shr_cnt.at[pl.ds(16384 + r * 256 + sid * L, L)])
        plsc.subcore_barrier()
        pltpu.sync_copy(shr_cnt.at[pl.ds(16384 + r * 256, 256)], red_cnt)

        def dj(j, st):
            carry, bacc, cntb = st
            cvec = red_cnt[pl.ds(j * L, L)]
            cum = plsc.cumsum(cvec) + carry
            m = cum < kprime
            bacc = bacc + jnp.sum(jnp.where(m, ones, zi))
            cntb = cntb + jnp.sum(jnp.where(m, cvec, zi))
            carry = carry + jnp.sum(cvec)
            return carry, bacc, cntb
        _, bacc, cntb = lax.fori_loop(
            0, 16, dj, (jnp.int32(0), jnp.int32(0), jnp.int32(0)))
        return bacc, kprime - cntb

    zero_hist()
    base = sid * EPT
    descs = [
        pltpu.async_copy(x_hbm.at[pl.ds(base + c * VCH, VCH)],
                         keybuf.at[pl.ds(c * VCH, VCH)], dsem.at[c])
        for c in range(NCH)
    ]
    for c in range(NCH):
        descs[c].wait()

        def r0(i, c=c):
            v = keybuf[pl.ds(c * VCH + i * L, L)]
            key = _fwd_key(v)
            byte = (key >> jnp.uint32(24)).astype(jnp.int32)
            plsc.addupdate_scatter(hcnt, [lane * 256 + byte], ones)
        plsc.parallel_loop(0, VCH // L, unroll=8)(r0)

    kprime = jnp.int32(K)
    b0, kprime = publish_and_decide(0, kprime)
    prefix = b0.astype(jnp.uint32)
    n = jnp.int32(EPT)
    sumb = zf

    for r in (1, 2, 3):
        zero_hist()
        shift_prefix = jnp.uint32(32 - 8 * r)
        byte_shift = jnp.uint32(24 - 8 * r)
        thr = prefix << shift_prefix
        nch = (n + L - 1) >> 4

        def rbody(i, st, shift_prefix=shift_prefix, byte_shift=byte_shift,
                  prefix=prefix, thr=thr, n=n):
            wp, sb = st
            kf = keybuf[pl.ds(i * L, L)]
            key = _fwd_key(kf)
            valid = (i * L + lane) < n
            act = valid & ((key >> shift_prefix) == prefix)
            below = valid & (key < thr)
            byte = ((key >> byte_shift)
                    & jnp.uint32(0xFF)).astype(jnp.int32)
            plsc.addupdate_scatter(hcnt, [lane * 256 + byte], ones,
                                   mask=act)
            sb = sb + jnp.where(below, kf, zf)
            plsc.store_compressed(keybuf.at[pl.ds(wp, L)], kf, mask=act)
            wp = wp + plsc.all_reduce_population_count(act)[0]
            return wp, sb

        n_new, sumb = plsc.parallel_loop(
            0, nch, carry=(jnp.int32(0), sumb), unroll=4)(rbody)
        br, kprime = publish_and_decide(r, kprime)
        prefix = prefix * jnp.uint32(256) + br.astype(jnp.uint32)
        n = n_new

    nch = (n + L - 1) >> 4

    def fbody(i, sb, prefix=prefix, n=n):
        kf = keybuf[pl.ds(i * L, L)]
        key = _fwd_key(kf)
        below = ((i * L + lane) < n) & (key < prefix)
        return sb + jnp.where(below, kf, zf)

    sumb = lax.fori_loop(0, nch, fbody, sumb)

    outv[...] = sumb
    pltpu.sync_copy(outv, shr_sum.at[pl.ds(sid * L, L)])
    plsc.subcore_barrier()

    @pl.when(sid == 0)
    def _():
        pltpu.sync_copy(shr_sum, sball)

        def sred(t, acc):
            return acc + sball[pl.ds(t * L, L)]
        sacc = lax.fori_loop(0, 16, sred, zf)
        tval = _inv_val(jnp.full((L,), prefix, jnp.uint32))
        total_below = jnp.sum(sacc)
        res = (total_below
               + kprime.astype(jnp.float32) * tval) / jnp.float32(K)
        outv[...] = res
        pltpu.sync_copy(outv, out_hbm)


def _build():
    mesh = plsc.VectorSubcoreMesh(
        core_axis_name="c", subcore_axis_name="s", num_cores=1,
        num_subcores=NT)
    return pl.kernel(
        _tail_body,
        out_type=jax.ShapeDtypeStruct((L,), jnp.float32),
        mesh=mesh,
        scratch_types=[
            pltpu.VMEM((EPT,), jnp.float32),
            pltpu.VMEM((16 * 256,), jnp.int32),
            pltpu.VMEM((256,), jnp.int32),
            pltpu.VMEM((L,), jnp.int32),
            pltpu.VMEM((L,), jnp.float32),
            pltpu.VMEM((16 * L,), jnp.float32),
            pltpu.SemaphoreType.DMA((16,)),
            pltpu.VMEM_SHARED((4 * 4096 + 4 * 256,), jnp.int32),
            pltpu.VMEM_SHARED((16 * L,), jnp.float32),
        ],
        compiler_params=pltpu.CompilerParams(needs_layout_passes=False),
    )


_tail_kernel = None


def kernel(portfolio_value):
    global _tail_kernel
    if _tail_kernel is None:
        _tail_kernel = jax.jit(_build())
    return _tail_kernel(portfolio_value)[0]
